# baseline (speedup 1.0000x reference)
import functools

import jax
import jax.numpy as jnp
from jax import lax
from jax.experimental import pallas as pl
from jax.experimental.pallas import tpu as pltpu
from jax.experimental.pallas import tpu_sc as plsc

VOCAB = 1000000
HIDDEN = 64
MAX_POS = 20
BATCH = 16384
EPS = 1e-05

NC = 2
NS = 16
NW = NC * NS
LANES = 16

ROWS = BATCH * MAX_POS
ROWS_PER_W = ROWS // NW
JBLK = 128
CHUNK = 256
NJ = CHUNK // JBLK
NCHUNK = ROWS_PER_W // CHUNK
NGRP = CHUNK // LANES
NV = HIDDEN // LANES
RSTRIDE = HIDDEN + 1
GBLK = 8


def _rsqrt_newton(v):
    i = lax.bitcast_convert_type(v, jnp.int32)
    i = jnp.int32(0x5F3759DF) - lax.shift_right_logical(i, 1)
    y = lax.bitcast_convert_type(i, jnp.float32)
    half = jnp.float32(0.5) * v
    for _ in range(2):
        y = y * (jnp.float32(1.5) - half * y * y)
    return y


def _body(ids_hbm, table_hbm, pos_hbm, gamma_hbm, beta_hbm, out_hbm,
          idx_v, bounce, rows_f, ostage, pos_v, gb_v, pos_s, gam_s, bet_s,
          gsem, osem):
    wid = lax.axis_index("s") * NC + lax.axis_index("c")
    base = wid * ROWS_PER_W

    pltpu.sync_copy(pos_hbm, pos_v)
    pltpu.sync_copy(gamma_hbm, gb_v.at[0])
    pltpu.sync_copy(beta_hbm, gb_v.at[1])

    @pl.loop(0, MAX_POS)
    def _fill_pos_smem(l):
        for k in range(NV):
            v = pos_v[l, pl.ds(LANES * k, LANES)]
            for t in range(LANES):
                pos_s[l, LANES * k + t] = v[t]

    for k in range(NV):
        vg = gb_v[0, pl.ds(LANES * k, LANES)]
        vb = gb_v[1, pl.ds(LANES * k, LANES)]
        for t in range(LANES):
            gam_s[LANES * k + t] = vg[t]
            bet_s[LANES * k + t] = vb[t]

    inv_h = jnp.float32(1.0 / HIDDEN)
    inv_hm1 = jnp.float32(1.0 / (HIDDEN - 1))
    eps = jnp.float32(EPS)
    iota = lax.iota(jnp.int32, LANES)
    iota65 = iota * RSTRIDE
    zf = jnp.zeros((LANES,), jnp.float32)

    def stage(c, b):
        i0 = base + c * CHUNK
        pltpu.sync_copy(ids_hbm.at[pl.ds(i0, CHUNK)], idx_v.at[b])
        for j in range(NJ):
            pltpu.async_copy(
                table_hbm.at[idx_v.at[b].at[pl.ds(j * JBLK, JBLK)]],
                bounce.at[b].at[pl.ds(j * JBLK, JBLK)],
                gsem[b])

    def drain_gathers(b):
        for j in range(NJ):
            pltpu.make_async_copy(
                table_hbm.at[idx_v.at[b].at[pl.ds(j * JBLK, JBLK)]],
                bounce.at[b].at[pl.ds(j * JBLK, JBLK)],
                gsem[b]).wait()

    def drain_out(b, l, b0):
        pltpu.make_async_copy(
            ostage.at[b],
            out_hbm.at[pl.ds(l * HIDDEN, HIDDEN)].at[:, pl.ds(b0, CHUNK)],
            osem[b]).wait()

    def compute(c, b):
        i0 = base + c * CHUNK
        l = i0 // BATCH
        bnc = bounce.at[b]
        ost = ostage.at[b]

        @pl.loop(0, CHUNK, unroll=4)
        def _restride(r):
            for k in range(NV):
                rows_f[pl.ds(r * RSTRIDE + LANES * k, LANES)] = (
                    bnc[r, pl.ds(LANES * k, LANES)])

        for blk in range(NGRP // GBLK):
            tset = [blk * GBLK + t for t in range(GBLK)]

            def p1(h, carry):
                ss, qq = carry
                p = pos_s[l, h]
                ss2, qq2 = [], []
                for n, t in enumerate(tset):
                    v = plsc.load_gather(
                        rows_f, [iota65 + (t * LANES * RSTRIDE + h)])
                    x = v + p
                    ost[h, pl.ds(t * LANES, LANES)] = x
                    ss2.append(ss[n] + x)
                    qq2.append(qq[n] + x * x)
                return tuple(ss2), tuple(qq2)

            zs = tuple(zf for _ in tset)
            ss, qq = lax.fori_loop(0, HIDDEN, p1, (zs, zs), unroll=4)

            means, invs = [], []
            for n in range(GBLK):
                mean = ss[n] * inv_h
                var = jnp.maximum((qq[n] - ss[n] * mean) * inv_hm1,
                                  jnp.float32(0.0))
                std = var * _rsqrt_newton(var)
                means.append(mean)
                invs.append(jnp.float32(1.0) / (std + eps))

            def p2(h, carry):
                g = gam_s[h]
                be = bet_s[h]
                for n, t in enumerate(tset):
                    x = ost[h, pl.ds(t * LANES, LANES)]
                    o = (x - means[n]) * (invs[n] * g) + be
                    ost[h, pl.ds(t * LANES, LANES)] = o
                return carry

            lax.fori_loop(0, HIDDEN, p2, 0, unroll=4)

    stage(0, 0)

    @pl.loop(0, NCHUNK, step=2)
    def _chunks(c0):
        for b in range(2):
            c = c0 + b

            @pl.when(c + 1 < NCHUNK)
            def _prefetch():
                stage(c + 1, 1 - b)

            @pl.when(c >= 2)
            def _drain_prev():
                ip = base + (c - 2) * CHUNK
                lp = ip // BATCH
                drain_out(b, lp, ip - lp * BATCH)

            drain_gathers(b)
            compute(c, b)
            i0 = base + c * CHUNK
            l = i0 // BATCH
            b0 = i0 - l * BATCH
            pltpu.async_copy(
                ostage.at[b],
                out_hbm.at[pl.ds(l * HIDDEN, HIDDEN)].at[:, pl.ds(b0, CHUNK)],
                osem[b])

    for b in range(2):
        ip = base + (NCHUNK - 2 + b) * CHUNK
        lp = ip // BATCH
        drain_out(b, lp, ip - lp * BATCH)


@functools.partial(
    pl.kernel,
    out_type=jax.ShapeDtypeStruct((MAX_POS * HIDDEN, BATCH), jnp.float32),
    mesh=plsc.VectorSubcoreMesh(core_axis_name="c", subcore_axis_name="s"),
    scratch_types=[
        pltpu.VMEM((2, CHUNK), jnp.int32),
        pltpu.VMEM((2, CHUNK, HIDDEN), jnp.float32),
        pltpu.VMEM((CHUNK * RSTRIDE,), jnp.float32),
        pltpu.VMEM((2, HIDDEN, CHUNK), jnp.float32),
        pltpu.VMEM((MAX_POS, HIDDEN), jnp.float32),
        pltpu.VMEM((2, HIDDEN), jnp.float32),
        pltpu.SMEM((MAX_POS, HIDDEN), jnp.float32),
        pltpu.SMEM((HIDDEN,), jnp.float32),
        pltpu.SMEM((HIDDEN,), jnp.float32),
        [pltpu.SemaphoreType.DMA, pltpu.SemaphoreType.DMA],
        [pltpu.SemaphoreType.DMA, pltpu.SemaphoreType.DMA],
    ],
    compiler_params=pltpu.CompilerParams(use_tc_tiling_on_sc=False,
                                         needs_layout_passes=False),
)
def _embed_ln(*args):
    _body(*args)


def kernel(input_ids, table, pos_table, gamma, beta):
    ids_t = input_ids.astype(jnp.int32).T.reshape(ROWS)
    out2 = _embed_ln(ids_t, table, pos_table, gamma, beta)
    return out2.reshape(MAX_POS, HIDDEN, BATCH).transpose(2, 0, 1)


# speedup vs baseline: 1.1445x; 1.1445x over previous
import functools

import jax
import jax.numpy as jnp
from jax import lax
from jax.experimental import pallas as pl
from jax.experimental.pallas import tpu as pltpu
from jax.experimental.pallas import tpu_sc as plsc

VOCAB = 1000000
HIDDEN = 64
MAX_POS = 20
BATCH = 16384
EPS = 1e-05

NC = 2
NS = 16
NW = NC * NS
LANES = 16

ROWS = BATCH * MAX_POS
ROWS_PER_W = ROWS // NW
JBLK = 128
CHUNK = 256
NJ = CHUNK // JBLK
NCHUNK = ROWS_PER_W // CHUNK
NGRP = CHUNK // LANES
NV = HIDDEN // LANES
RSTRIDE = HIDDEN + 1
GBLK = 8


def _rsqrt_newton(v):
    i = lax.bitcast_convert_type(v, jnp.int32)
    i = jnp.int32(0x5F3759DF) - lax.shift_right_logical(i, 1)
    y = lax.bitcast_convert_type(i, jnp.float32)
    half = jnp.float32(0.5) * v
    for _ in range(2):
        y = y * (jnp.float32(1.5) - half * y * y)
    return y


def _body(ids_hbm, table_hbm, pos_hbm, gamma_hbm, beta_hbm, out_hbm,
          idx_v, bounce, rows_f, ostage, pos_v, gb_v, pos_s, gam_s, bet_s,
          gsem, osem):
    wid = lax.axis_index("s") * NC + lax.axis_index("c")
    base = wid * ROWS_PER_W

    pltpu.sync_copy(pos_hbm, pos_v)
    pltpu.sync_copy(gamma_hbm, gb_v.at[0])
    pltpu.sync_copy(beta_hbm, gb_v.at[1])

    @pl.loop(0, MAX_POS)
    def _fill_pos_smem(l):
        for k in range(NV):
            v = pos_v[l, pl.ds(LANES * k, LANES)]
            for t in range(LANES):
                pos_s[l, LANES * k + t] = v[t]

    for k in range(NV):
        vg = gb_v[0, pl.ds(LANES * k, LANES)]
        vb = gb_v[1, pl.ds(LANES * k, LANES)]
        for t in range(LANES):
            gam_s[LANES * k + t] = vg[t]
            bet_s[LANES * k + t] = vb[t]

    inv_h = jnp.float32(1.0 / HIDDEN)
    inv_hm1 = jnp.float32(1.0 / (HIDDEN - 1))
    eps = jnp.float32(EPS)
    iota = lax.iota(jnp.int32, LANES)
    iota65 = iota * RSTRIDE
    zf = jnp.zeros((LANES,), jnp.float32)

    def stage(c, b):
        i0 = base + c * CHUNK
        pltpu.sync_copy(ids_hbm.at[pl.ds(i0, CHUNK)], idx_v.at[b])
        for j in range(NJ):
            pltpu.async_copy(
                table_hbm.at[idx_v.at[b].at[pl.ds(j * JBLK, JBLK)]],
                bounce.at[b].at[pl.ds(j * JBLK, JBLK)],
                gsem[b])

    def drain_gathers(b):
        for j in range(NJ):
            pltpu.make_async_copy(
                table_hbm.at[idx_v.at[b].at[pl.ds(j * JBLK, JBLK)]],
                bounce.at[b].at[pl.ds(j * JBLK, JBLK)],
                gsem[b]).wait()

    def drain_out(b, l, b0):
        pltpu.make_async_copy(
            ostage.at[b],
            out_hbm.at[pl.ds(l * HIDDEN, HIDDEN)].at[:, pl.ds(b0, CHUNK)],
            osem[b]).wait()

    def compute(c, b):
        i0 = base + c * CHUNK
        l = i0 // BATCH
        bnc = bounce.at[b]
        ost = ostage.at[b]

        @pl.loop(0, 0, unroll=4)
        def _restride(r):
            for k in range(NV):
                rows_f[pl.ds(r * RSTRIDE + LANES * k, LANES)] = (
                    bnc[r, pl.ds(LANES * k, LANES)])

        for blk in range(NGRP // GBLK):
            tset = [blk * GBLK + t for t in range(GBLK)]

            def p1(h, carry):
                ss, qq = carry
                p = pos_s[l, h]
                ss2, qq2 = [], []
                for n, t in enumerate(tset):
                    v = plsc.load_gather(
                        rows_f, [iota65 + (t * LANES * RSTRIDE + h)])
                    x = v + p
                    ost[h, pl.ds(t * LANES, LANES)] = x
                    ss2.append(ss[n] + x)
                    qq2.append(qq[n] + x * x)
                return tuple(ss2), tuple(qq2)

            zs = tuple(zf for _ in tset)
            ss, qq = lax.fori_loop(0, HIDDEN, p1, (zs, zs), unroll=4)

            means, invs = [], []
            for n in range(GBLK):
                mean = ss[n] * inv_h
                var = jnp.maximum((qq[n] - ss[n] * mean) * inv_hm1,
                                  jnp.float32(0.0))
                std = var * _rsqrt_newton(var)
                means.append(mean)
                invs.append(jnp.float32(1.0) / (std + eps))

            def p2(h, carry):
                g = gam_s[h]
                be = bet_s[h]
                for n, t in enumerate(tset):
                    x = ost[h, pl.ds(t * LANES, LANES)]
                    o = (x - means[n]) * (invs[n] * g) + be
                    ost[h, pl.ds(t * LANES, LANES)] = o
                return carry

            lax.fori_loop(0, HIDDEN, p2, 0, unroll=4)

    stage(0, 0)

    @pl.loop(0, NCHUNK, step=2)
    def _chunks(c0):
        for b in range(2):
            c = c0 + b

            @pl.when(c + 1 < NCHUNK)
            def _prefetch():
                stage(c + 1, 1 - b)

            @pl.when(c >= 2)
            def _drain_prev():
                ip = base + (c - 2) * CHUNK
                lp = ip // BATCH
                drain_out(b, lp, ip - lp * BATCH)

            drain_gathers(b)
            compute(c, b)
            i0 = base + c * CHUNK
            l = i0 // BATCH
            b0 = i0 - l * BATCH
            pltpu.async_copy(
                ostage.at[b],
                out_hbm.at[pl.ds(l * HIDDEN, HIDDEN)].at[:, pl.ds(b0, CHUNK)],
                osem[b])

    for b in range(2):
        ip = base + (NCHUNK - 2 + b) * CHUNK
        lp = ip // BATCH
        drain_out(b, lp, ip - lp * BATCH)


@functools.partial(
    pl.kernel,
    out_type=jax.ShapeDtypeStruct((MAX_POS * HIDDEN, BATCH), jnp.float32),
    mesh=plsc.VectorSubcoreMesh(core_axis_name="c", subcore_axis_name="s"),
    scratch_types=[
        pltpu.VMEM((2, CHUNK), jnp.int32),
        pltpu.VMEM((2, CHUNK, HIDDEN), jnp.float32),
        pltpu.VMEM((CHUNK * RSTRIDE,), jnp.float32),
        pltpu.VMEM((2, HIDDEN, CHUNK), jnp.float32),
        pltpu.VMEM((MAX_POS, HIDDEN), jnp.float32),
        pltpu.VMEM((2, HIDDEN), jnp.float32),
        pltpu.SMEM((MAX_POS, HIDDEN), jnp.float32),
        pltpu.SMEM((HIDDEN,), jnp.float32),
        pltpu.SMEM((HIDDEN,), jnp.float32),
        [pltpu.SemaphoreType.DMA, pltpu.SemaphoreType.DMA],
        [pltpu.SemaphoreType.DMA, pltpu.SemaphoreType.DMA],
    ],
    compiler_params=pltpu.CompilerParams(use_tc_tiling_on_sc=False,
                                         needs_layout_passes=False),
)
def _embed_ln(*args):
    _body(*args)


def kernel(input_ids, table, pos_table, gamma, beta):
    ids_t = input_ids.astype(jnp.int32).T.reshape(ROWS)
    out2 = _embed_ln(ids_t, table, pos_table, gamma, beta)
    return out2.reshape(MAX_POS, HIDDEN, BATCH).transpose(2, 0, 1)


# speedup vs baseline: 1.1474x; 1.0025x over previous
import functools

import jax
import jax.numpy as jnp
from jax import lax
from jax.experimental import pallas as pl
from jax.experimental.pallas import tpu as pltpu
from jax.experimental.pallas import tpu_sc as plsc

VOCAB = 1000000
HIDDEN = 64
MAX_POS = 20
BATCH = 16384
EPS = 1e-05

NC = 2
NS = 16
NW = NC * NS
LANES = 16

ROWS = BATCH * MAX_POS
ROWS_PER_W = ROWS // NW
JBLK = 128
CHUNK = 256
NJ = CHUNK // JBLK
NCHUNK = ROWS_PER_W // CHUNK
NGRP = CHUNK // LANES
NV = HIDDEN // LANES
GBLK = 8


def _rsqrt_newton(v):
    i = lax.bitcast_convert_type(v, jnp.int32)
    i = jnp.int32(0x5F3759DF) - lax.shift_right_logical(i, 1)
    y = lax.bitcast_convert_type(i, jnp.float32)
    half = jnp.float32(0.5) * v
    for _ in range(2):
        y = y * (jnp.float32(1.5) - half * y * y)
    return y


def _body(ids_hbm, table_hbm, pos_hbm, gamma_hbm, beta_hbm, out_hbm,
          idx_v, bounce, ostage, pos_v, gb_v, gam_s, bet_s,
          gsem, osem):
    wid = lax.axis_index("s") * NC + lax.axis_index("c")
    base = wid * ROWS_PER_W

    pltpu.sync_copy(pos_hbm, pos_v)
    pltpu.sync_copy(gamma_hbm, gb_v.at[0])
    pltpu.sync_copy(beta_hbm, gb_v.at[1])

    for k in range(NV):
        vg = gb_v[0, pl.ds(LANES * k, LANES)]
        vb = gb_v[1, pl.ds(LANES * k, LANES)]
        for t in range(LANES):
            gam_s[LANES * k + t] = vg[t]
            bet_s[LANES * k + t] = vb[t]

    inv_h = jnp.float32(1.0 / HIDDEN)
    inv_hm1 = jnp.float32(1.0 / (HIDDEN - 1))
    eps = jnp.float32(EPS)
    iota = lax.iota(jnp.int32, LANES)
    zf = jnp.zeros((LANES,), jnp.float32)

    def stage(c, b):
        i0 = base + c * CHUNK
        pltpu.sync_copy(ids_hbm.at[pl.ds(i0, CHUNK)], idx_v.at[b])
        for j in range(NJ):
            pltpu.async_copy(
                table_hbm.at[idx_v.at[b].at[pl.ds(j * JBLK, JBLK)]],
                bounce.at[b].at[pl.ds(j * JBLK, JBLK)],
                gsem[b])

    def drain_gathers(b):
        for j in range(NJ):
            pltpu.make_async_copy(
                table_hbm.at[idx_v.at[b].at[pl.ds(j * JBLK, JBLK)]],
                bounce.at[b].at[pl.ds(j * JBLK, JBLK)],
                gsem[b]).wait()

    def drain_out(b, l, b0):
        pltpu.make_async_copy(
            ostage.at[b],
            out_hbm.at[pl.ds(l * HIDDEN, HIDDEN)].at[:, pl.ds(b0, CHUNK)],
            osem[b]).wait()

    def compute(c, b):
        i0 = base + c * CHUNK
        l = i0 // BATCH
        bnc = bounce.at[b]
        ost = ostage.at[b]
        lvec = jnp.broadcast_to(l, (LANES,)).astype(jnp.int32)

        for blk in range(NGRP // GBLK):
            tset = [blk * GBLK + t for t in range(GBLK)]

            def p1(h, carry):
                ss, qq = carry
                hd = (h & ~15) + lax.bitwise_and(h + iota, jnp.int32(15))
                ph = plsc.load_gather(pos_v, [lvec, hd])
                ss2, qq2 = [], []
                for n, t in enumerate(tset):
                    cvec = iota + t * LANES
                    v = plsc.load_gather(bnc, [cvec, hd])
                    x = v + ph
                    plsc.store_scatter(ost, [hd, cvec], x)
                    ss2.append(ss[n] + x)
                    qq2.append(qq[n] + x * x)
                return tuple(ss2), tuple(qq2)

            zs = tuple(zf for _ in tset)
            ss, qq = lax.fori_loop(0, HIDDEN, p1, (zs, zs), unroll=2)

            means, invs = [], []
            for n in range(GBLK):
                mean = ss[n] * inv_h
                var = jnp.maximum((qq[n] - ss[n] * mean) * inv_hm1,
                                  jnp.float32(0.0))
                std = var * _rsqrt_newton(var)
                means.append(mean)
                invs.append(jnp.float32(1.0) / (std + eps))

            def p2(h, carry):
                g = gam_s[h]
                be = bet_s[h]
                for n, t in enumerate(tset):
                    x = ost[h, pl.ds(t * LANES, LANES)]
                    o = (x - means[n]) * (invs[n] * g) + be
                    ost[h, pl.ds(t * LANES, LANES)] = o
                return carry

            lax.fori_loop(0, HIDDEN, p2, 0, unroll=2)

    stage(0, 0)

    @pl.loop(0, NCHUNK, step=2)
    def _chunks(c0):
        for b in range(2):
            c = c0 + b

            @pl.when(c + 1 < NCHUNK)
            def _prefetch():
                stage(c + 1, 1 - b)

            @pl.when(c >= 2)
            def _drain_prev():
                ip = base + (c - 2) * CHUNK
                lp = ip // BATCH
                drain_out(b, lp, ip - lp * BATCH)

            drain_gathers(b)
            compute(c, b)
            i0 = base + c * CHUNK
            l = i0 // BATCH
            b0 = i0 - l * BATCH
            pltpu.async_copy(
                ostage.at[b],
                out_hbm.at[pl.ds(l * HIDDEN, HIDDEN)].at[:, pl.ds(b0, CHUNK)],
                osem[b])

    for b in range(2):
        ip = base + (NCHUNK - 2 + b) * CHUNK
        lp = ip // BATCH
        drain_out(b, lp, ip - lp * BATCH)


@functools.partial(
    pl.kernel,
    out_type=jax.ShapeDtypeStruct((MAX_POS * HIDDEN, BATCH), jnp.float32),
    mesh=plsc.VectorSubcoreMesh(core_axis_name="c", subcore_axis_name="s"),
    scratch_types=[
        pltpu.VMEM((2, CHUNK), jnp.int32),
        pltpu.VMEM((2, CHUNK, HIDDEN), jnp.float32),
        pltpu.VMEM((2, HIDDEN, CHUNK), jnp.float32),
        pltpu.VMEM((MAX_POS, HIDDEN), jnp.float32),
        pltpu.VMEM((2, HIDDEN), jnp.float32),
        pltpu.SMEM((HIDDEN,), jnp.float32),
        pltpu.SMEM((HIDDEN,), jnp.float32),
        [pltpu.SemaphoreType.DMA, pltpu.SemaphoreType.DMA],
        [pltpu.SemaphoreType.DMA, pltpu.SemaphoreType.DMA],
    ],
    compiler_params=pltpu.CompilerParams(use_tc_tiling_on_sc=False,
                                         needs_layout_passes=False),
)
def _embed_ln(*args):
    _body(*args)


def kernel(input_ids, table, pos_table, gamma, beta):
    ids_t = input_ids.astype(jnp.int32).T.reshape(ROWS)
    out2 = _embed_ln(ids_t, table, pos_table, gamma, beta)
    return out2.reshape(MAX_POS, HIDDEN, BATCH).transpose(2, 0, 1)


# speedup vs baseline: 1.2677x; 1.1049x over previous
import functools

import jax
import jax.numpy as jnp
from jax import lax
from jax.experimental import pallas as pl
from jax.experimental.pallas import tpu as pltpu
from jax.experimental.pallas import tpu_sc as plsc

VOCAB = 1000000
HIDDEN = 64
MAX_POS = 20
BATCH = 16384
EPS = 1e-05

NC = 2
NS = 16
NW = NC * NS
LANES = 16

ROWS = BATCH * MAX_POS
ROWS_PER_W = ROWS // NW
JBLK = 128
CHUNK = 256
NJ = CHUNK // JBLK
NCHUNK = ROWS_PER_W // CHUNK
NGRP = CHUNK // LANES
NV = HIDDEN // LANES
GBLK = 8


def _rsqrt_newton(v):
    i = lax.bitcast_convert_type(v, jnp.int32)
    i = jnp.int32(0x5F3759DF) - lax.shift_right_logical(i, 1)
    y = lax.bitcast_convert_type(i, jnp.float32)
    half = jnp.float32(0.5) * v
    for _ in range(2):
        y = y * (jnp.float32(1.5) - half * y * y)
    return y


def _body(ids_hbm, table_hbm, pos_hbm, gamma_hbm, beta_hbm, out_hbm,
          idx_v, bounce, ostage, pos_v, gb_v, gam_s, bet_s,
          gsem, osem):
    wid = lax.axis_index("s") * NC + lax.axis_index("c")
    base = wid * ROWS_PER_W

    pltpu.sync_copy(pos_hbm, pos_v)
    pltpu.sync_copy(gamma_hbm, gb_v.at[0])
    pltpu.sync_copy(beta_hbm, gb_v.at[1])

    for k in range(NV):
        vg = gb_v[0, pl.ds(LANES * k, LANES)]
        vb = gb_v[1, pl.ds(LANES * k, LANES)]
        for t in range(LANES):
            gam_s[LANES * k + t] = vg[t]
            bet_s[LANES * k + t] = vb[t]

    inv_h = jnp.float32(1.0 / HIDDEN)
    inv_hm1 = jnp.float32(1.0 / (HIDDEN - 1))
    eps = jnp.float32(EPS)
    iota = lax.iota(jnp.int32, LANES)
    zf = jnp.zeros((LANES,), jnp.float32)

    def stage(c, b):
        i0 = base + c * CHUNK
        pltpu.sync_copy(ids_hbm.at[pl.ds(i0, CHUNK)], idx_v.at[b])
        for j in range(NJ):
            pltpu.async_copy(
                table_hbm.at[idx_v.at[b].at[pl.ds(j * JBLK, JBLK)]],
                bounce.at[b].at[pl.ds(j * JBLK, JBLK)],
                gsem[b])

    def drain_gathers(b):
        for j in range(NJ):
            pltpu.make_async_copy(
                table_hbm.at[idx_v.at[b].at[pl.ds(j * JBLK, JBLK)]],
                bounce.at[b].at[pl.ds(j * JBLK, JBLK)],
                gsem[b]).wait()

    def drain_out(b, l, b0):
        pltpu.make_async_copy(
            ostage.at[b],
            out_hbm.at[pl.ds(l * (HIDDEN // 8), HIDDEN // 8)]
                   .at[:, pl.ds(b0 // 128, CHUNK // 128)],
            osem[b]).wait()

    def compute(c, b):
        i0 = base + c * CHUNK
        l = i0 // BATCH
        bnc = bounce.at[b]
        ost = ostage.at[b]
        lvec = jnp.broadcast_to(l, (LANES,)).astype(jnp.int32)
        jc = [jnp.broadcast_to(jnp.int32(t * LANES // 128), (LANES,))
              for t in range(NGRP)]

        for blk in range(NGRP // GBLK):
            tset = [blk * GBLK + t for t in range(GBLK)]

            def p1(h, carry):
                ss, qq = carry
                hd = (h & ~15) + lax.bitwise_and(h + iota, jnp.int32(15))
                ph = plsc.load_gather(pos_v, [lvec, hd])
                hi = lax.shift_right_logical(hd, 3)
                hr = lax.bitwise_and(hd, jnp.int32(7))
                ss2, qq2 = [], []
                for n, t in enumerate(tset):
                    cvec = iota + t * LANES
                    v = plsc.load_gather(bnc, [cvec, hd])
                    x = v + ph
                    plsc.store_scatter(
                        ost, [hi, jc[t], hr, iota + (t * LANES % 128)], x)
                    ss2.append(ss[n] + x)
                    qq2.append(qq[n] + x * x)
                return tuple(ss2), tuple(qq2)

            zs = tuple(zf for _ in tset)
            ss, qq = lax.fori_loop(0, HIDDEN, p1, (zs, zs), unroll=2)

            means, invs = [], []
            for n in range(GBLK):
                mean = ss[n] * inv_h
                var = jnp.maximum((qq[n] - ss[n] * mean) * inv_hm1,
                                  jnp.float32(0.0))
                std = var * _rsqrt_newton(var)
                means.append(mean)
                invs.append(jnp.float32(1.0) / (std + eps))

            def p2(h, carry):
                g = gam_s[h]
                be = bet_s[h]
                hi = h // 8
                hr = h % 8
                for n, t in enumerate(tset):
                    sl = (hi, t * LANES // 128, hr,
                          pl.ds(t * LANES % 128, LANES))
                    x = ost[sl]
                    o = (x - means[n]) * (invs[n] * g) + be
                    ost[sl] = o
                return carry

            lax.fori_loop(0, HIDDEN, p2, 0, unroll=2)

    stage(0, 0)

    @pl.loop(0, NCHUNK, step=2)
    def _chunks(c0):
        for b in range(2):
            c = c0 + b

            @pl.when(c + 1 < NCHUNK)
            def _prefetch():
                stage(c + 1, 1 - b)

            @pl.when(c >= 2)
            def _drain_prev():
                ip = base + (c - 2) * CHUNK
                lp = ip // BATCH
                drain_out(b, lp, ip - lp * BATCH)

            drain_gathers(b)
            compute(c, b)
            i0 = base + c * CHUNK
            l = i0 // BATCH
            b0 = i0 - l * BATCH
            pltpu.async_copy(
                ostage.at[b],
                out_hbm.at[pl.ds(l * (HIDDEN // 8), HIDDEN // 8)]
                       .at[:, pl.ds(b0 // 128, CHUNK // 128)],
                osem[b])

    for b in range(2):
        ip = base + (NCHUNK - 2 + b) * CHUNK
        lp = ip // BATCH
        drain_out(b, lp, ip - lp * BATCH)


@functools.partial(
    pl.kernel,
    out_type=jax.ShapeDtypeStruct(
        (MAX_POS * HIDDEN // 8, BATCH // 128, 8, 128), jnp.float32),
    mesh=plsc.VectorSubcoreMesh(core_axis_name="c", subcore_axis_name="s"),
    scratch_types=[
        pltpu.VMEM((2, CHUNK), jnp.int32),
        pltpu.VMEM((2, CHUNK, HIDDEN), jnp.float32),
        pltpu.VMEM((2, HIDDEN // 8, CHUNK // 128, 8, 128), jnp.float32),
        pltpu.VMEM((MAX_POS, HIDDEN), jnp.float32),
        pltpu.VMEM((2, HIDDEN), jnp.float32),
        pltpu.SMEM((HIDDEN,), jnp.float32),
        pltpu.SMEM((HIDDEN,), jnp.float32),
        [pltpu.SemaphoreType.DMA, pltpu.SemaphoreType.DMA],
        [pltpu.SemaphoreType.DMA, pltpu.SemaphoreType.DMA],
    ],
    compiler_params=pltpu.CompilerParams(use_tc_tiling_on_sc=False,
                                         needs_layout_passes=False),
)
def _embed_ln(*args):
    _body(*args)


def kernel(input_ids, table, pos_table, gamma, beta):
    ids_t = input_ids.astype(jnp.int32).T.reshape(ROWS)
    out4 = _embed_ln(ids_t, table, pos_table, gamma, beta)
    out2 = out4.transpose(0, 2, 1, 3).reshape(MAX_POS * HIDDEN, BATCH)
    return out2.reshape(MAX_POS, HIDDEN, BATCH).transpose(2, 0, 1)
